# Initial kernel scaffold; baseline (speedup 1.0000x reference)
import jax
import jax.numpy as jnp
from jax.experimental import pallas as pl


def kernel(x_Patient, x_Admission, params, ei_Patient_Admission, ei_Admission_Patient, ei_Admission_Diagnosis, ei_Diagnosis_Admission, ei_Admission_Medication, ei_Medication_Admission, ei_Admission_Procedure, ei_Procedure_Admission, ei_Admission_LabTest, ei_LabTest_Admission):
    raise NotImplementedError("write your pallas kernel here")


# speedup vs baseline: 1.0233x; 1.0233x over previous
import functools

import jax
import jax.numpy as jnp
from jax.experimental import pallas as pl
from jax.experimental.pallas import tpu as pltpu

H = 128
OUT = 64
NODE_TYPES = ['Patient', 'Admission', 'Diagnosis', 'Medication', 'Procedure', 'LabTest']
N_NODES = {'Patient': 20000, 'Admission': 40000, 'Diagnosis': 2000, 'Medication': 1000, 'Procedure': 2000, 'LabTest': 500}
EMB_TYPES = ['Diagnosis', 'Medication', 'Procedure', 'LabTest']
EDGE_TYPES = [('Patient', 'Admission', 40000), ('Admission', 'Patient', 40000), ('Admission', 'Diagnosis', 80000), ('Diagnosis', 'Admission', 80000), ('Admission', 'Medication', 60000), ('Medication', 'Admission', 60000), ('Admission', 'Procedure', 40000), ('Procedure', 'Admission', 40000), ('Admission', 'LabTest', 80000), ('LabTest', 'Admission', 80000)]

_EPS = 1e-5


def _rowblk(n):
    if n % 2000 == 0:
        return 2000
    if n % 1000 == 0:
        return 1000
    return 500


def _dot_t(x, w):
    return jax.lax.dot_general(x, w, (((1,), (1,)), ((), ())),
                               preferred_element_type=jnp.float32)


def _enc_body(nsteps, x_ref, w_ref, b_ref, y_ref, st_ref, acc_ref):
    y = _dot_t(x_ref[...], w_ref[...]) + b_ref[...]
    y_ref[...] = y
    i = pl.program_id(0)

    @pl.when(i == 0)
    def _():
        acc_ref[...] = jnp.zeros_like(acc_ref)

    acc_ref[0, :] += jnp.sum(y, axis=0)
    acc_ref[1, :] += jnp.sum(y * y, axis=0)

    @pl.when(i == nsteps - 1)
    def _():
        st_ref[...] = acc_ref[...]


def _encode(x, w, b):
    n, raw = x.shape
    blk = _rowblk(n)
    nsteps = n // blk
    y, st = pl.pallas_call(
        functools.partial(_enc_body, nsteps),
        grid=(nsteps,),
        in_specs=[
            pl.BlockSpec((blk, raw), lambda i: (i, 0)),
            pl.BlockSpec((H, raw), lambda i: (0, 0)),
            pl.BlockSpec((1, H), lambda i: (0, 0)),
        ],
        out_specs=[
            pl.BlockSpec((blk, H), lambda i: (i, 0)),
            pl.BlockSpec((8, H), lambda i: (0, 0)),
        ],
        out_shape=[
            jax.ShapeDtypeStruct((n, H), jnp.float32),
            jax.ShapeDtypeStruct((8, H), jnp.float32),
        ],
        scratch_shapes=[pltpu.VMEM((8, H), jnp.float32)],
    )(x, w, b.reshape(1, H))
    return y, st


def _bn_body(n, relu, x_ref, st_ref, g_ref, be_ref, o_ref):
    mu = st_ref[0, :] / n
    var = st_ref[1, :] / n - mu * mu
    scale = jax.lax.rsqrt(var + _EPS) * g_ref[0, :]
    y = (x_ref[...] - mu[None, :]) * scale[None, :] + be_ref[...]
    if relu:
        y = jnp.maximum(y, 0.0)
    o_ref[...] = y


def _bn_apply(x, st, g, be, relu):
    n = x.shape[0]
    blk = _rowblk(n)
    return pl.pallas_call(
        functools.partial(_bn_body, float(n), relu),
        grid=(n // blk,),
        in_specs=[
            pl.BlockSpec((blk, H), lambda i: (i, 0)),
            pl.BlockSpec((8, H), lambda i: (0, 0)),
            pl.BlockSpec((1, H), lambda i: (0, 0)),
            pl.BlockSpec((1, H), lambda i: (0, 0)),
        ],
        out_specs=pl.BlockSpec((blk, H), lambda i: (i, 0)),
        out_shape=jax.ShapeDtypeStruct((n, H), jnp.float32),
    )(x, st, g.reshape(1, H), be.reshape(1, H))


def _combine_body(k, nsteps, want_stats, *refs):
    sums = refs[0:k]
    cnts = refs[k:2 * k]
    hd_ref = refs[2 * k]
    wls = refs[2 * k + 1:3 * k + 1]
    bls = refs[3 * k + 1:4 * k + 1]
    wrs = refs[4 * k + 1:5 * k + 1]
    y_ref = refs[5 * k + 1]
    hd = hd_ref[...]
    y = None
    for j in range(k):
        cnt = cnts[j][..., 0:1]
        inv = 1.0 / jnp.maximum(cnt, 1.0)
        mean = sums[j][...] * inv
        r = _dot_t(mean, wls[j][...]) + bls[j][...] + _dot_t(hd, wrs[j][...])
        y = r if y is None else jnp.maximum(y, r)
    y_ref[...] = y
    if want_stats:
        st_ref, acc_ref = refs[5 * k + 2], refs[5 * k + 3]
        i = pl.program_id(0)

        @pl.when(i == 0)
        def _():
            acc_ref[...] = jnp.zeros_like(acc_ref)

        acc_ref[0, :] += jnp.sum(y, axis=0)
        acc_ref[1, :] += jnp.sum(y * y, axis=0)

        @pl.when(i == nsteps - 1)
        def _():
            st_ref[...] = acc_ref[...]


def _combine(sums, cnts, hd, wls, bls, wrs, want_stats):
    k = len(sums)
    n = hd.shape[0]
    blk = _rowblk(n)
    nsteps = n // blk
    in_specs = (
        [pl.BlockSpec((blk, H), lambda i: (i, 0))] * k
        + [pl.BlockSpec((blk, 32), lambda i: (i, 0))] * k
        + [pl.BlockSpec((blk, H), lambda i: (i, 0))]
        + [pl.BlockSpec((H, H), lambda i: (0, 0))] * k
        + [pl.BlockSpec((1, H), lambda i: (0, 0))] * k
        + [pl.BlockSpec((H, H), lambda i: (0, 0))] * k
    )
    out_specs = [pl.BlockSpec((blk, H), lambda i: (i, 0))]
    out_shape = [jax.ShapeDtypeStruct((n, H), jnp.float32)]
    scratch = []
    if want_stats:
        out_specs.append(pl.BlockSpec((8, H), lambda i: (0, 0)))
        out_shape.append(jax.ShapeDtypeStruct((8, H), jnp.float32))
        scratch.append(pltpu.VMEM((8, H), jnp.float32))
    args = list(sums) + list(cnts) + [hd] + list(wls) \
        + [b.reshape(1, H) for b in bls] + list(wrs)
    res = pl.pallas_call(
        functools.partial(_combine_body, k, nsteps, want_stats),
        grid=(nsteps,),
        in_specs=in_specs,
        out_specs=out_specs,
        out_shape=out_shape,
        scratch_shapes=scratch,
    )(*args)
    return res if want_stats else (res[0], None)


def _final_body(x_ref, w_ref, b_ref, o_ref):
    x = jnp.maximum(x_ref[...], 0.0)
    o_ref[...] = _dot_t(x, w_ref[...]) + b_ref[...]


def _final(x, w, b):
    n = x.shape[0]
    blk = _rowblk(n)
    return pl.pallas_call(
        _final_body,
        grid=(n // blk,),
        in_specs=[
            pl.BlockSpec((blk, H), lambda i: (i, 0)),
            pl.BlockSpec((OUT, H), lambda i: (0, 0)),
            pl.BlockSpec((1, OUT), lambda i: (0, 0)),
        ],
        out_specs=pl.BlockSpec((blk, OUT), lambda i: (i, 0)),
        out_shape=jax.ShapeDtypeStruct((n, OUT), jnp.float32),
    )(x, w, b.reshape(1, OUT))


def _segment_sums(h, eis):
    out = {}
    for s, d, e in EDGE_TYPES:
        ei = eis[(s, d)]
        msgs = h[s][ei[0]]
        out[(s, d)] = jax.ops.segment_sum(msgs, ei[1], num_segments=N_NODES[d])
    return out


def _segment_counts(eis):
    out = {}
    for s, d, e in EDGE_TYPES:
        ei = eis[(s, d)]
        cnt = jax.ops.segment_sum(jnp.ones((e,), jnp.float32), ei[1],
                                  num_segments=N_NODES[d])
        out[(s, d)] = jnp.broadcast_to(cnt[:, None], (N_NODES[d], 32))
    return out


def _hetero_layer(h, convp, eis, cnts, want_stats):
    sums = _segment_sums(h, eis)
    incoming = {}
    for s, d, _ in EDGE_TYPES:
        incoming.setdefault(d, []).append((s, d))
    out = {}
    for d, keys in incoming.items():
        p = [convp[s + '__' + dd] for s, dd in keys]
        y, st = _combine([sums[k] for k in keys], [cnts[k] for k in keys],
                         h[d], [q['wl'] for q in p], [q['bl'] for q in p],
                         [q['wr'] for q in p], want_stats)
        out[d] = (y, st)
    return out


def kernel(x_Patient, x_Admission, params, ei_Patient_Admission, ei_Admission_Patient, ei_Admission_Diagnosis, ei_Diagnosis_Admission, ei_Admission_Medication, ei_Medication_Admission, ei_Admission_Procedure, ei_Procedure_Admission, ei_Admission_LabTest, ei_LabTest_Admission):
    eis = {
        ('Patient', 'Admission'): ei_Patient_Admission,
        ('Admission', 'Patient'): ei_Admission_Patient,
        ('Admission', 'Diagnosis'): ei_Admission_Diagnosis,
        ('Diagnosis', 'Admission'): ei_Diagnosis_Admission,
        ('Admission', 'Medication'): ei_Admission_Medication,
        ('Medication', 'Admission'): ei_Medication_Admission,
        ('Admission', 'Procedure'): ei_Admission_Procedure,
        ('Procedure', 'Admission'): ei_Procedure_Admission,
        ('Admission', 'LabTest'): ei_Admission_LabTest,
        ('LabTest', 'Admission'): ei_LabTest_Admission,
    }
    enc = params['enc']
    h = {}
    yp, stp = _encode(x_Patient, enc['pat_w'], enc['pat_b'])
    h['Patient'] = _bn_apply(yp, stp, enc['pat_g'], enc['pat_be'], relu=False)
    ya, sta = _encode(x_Admission, enc['adm_w'], enc['adm_b'])
    h['Admission'] = _bn_apply(ya, sta, enc['adm_g'], enc['adm_be'], relu=False)
    for t in EMB_TYPES:
        h[t] = enc['emb_' + t]

    cnts = _segment_counts(eis)

    for L, bn in (('conv1', 'bn1'), ('conv2', 'bn2')):
        res = _hetero_layer(h, params[L], eis, cnts, want_stats=True)
        h = {t: _bn_apply(res[t][0], res[t][1], params[bn][t]['g'],
                          params[bn][t]['b'], relu=True)
             for t in NODE_TYPES}

    res = _hetero_layer(h, params['conv3'], eis, cnts, want_stats=False)
    out = [_final(res[t][0], params['lin'][t]['w'], params['lin'][t]['b'])
           for t in NODE_TYPES]
    return jnp.concatenate(out, axis=0)


# speedup vs baseline: 1.1514x; 1.1252x over previous
import functools

import jax
import jax.numpy as jnp
from jax import lax
from jax.experimental import pallas as pl
from jax.experimental.pallas import tpu as pltpu
from jax.experimental.pallas import tpu_sc as plsc

H = 128
OUT = 64
NODE_TYPES = ['Patient', 'Admission', 'Diagnosis', 'Medication', 'Procedure', 'LabTest']
N_NODES = {'Patient': 20000, 'Admission': 40000, 'Diagnosis': 2000, 'Medication': 1000, 'Procedure': 2000, 'LabTest': 500}
EMB_TYPES = ['Diagnosis', 'Medication', 'Procedure', 'LabTest']
EDGE_TYPES = [('Patient', 'Admission', 40000), ('Admission', 'Patient', 40000), ('Admission', 'Diagnosis', 80000), ('Diagnosis', 'Admission', 80000), ('Admission', 'Medication', 60000), ('Medication', 'Admission', 60000), ('Admission', 'Procedure', 40000), ('Procedure', 'Admission', 40000), ('Admission', 'LabTest', 80000), ('LabTest', 'Admission', 80000)]

_EPS = 1e-5


def _rowblk(n):
    if n % 2000 == 0:
        return 2000
    if n % 1000 == 0:
        return 1000
    return n


def _dot_t(x, w):
    return jax.lax.dot_general(x, w, (((1,), (1,)), ((), ())),
                               preferred_element_type=jnp.float32)


def _enc_body(nsteps, x_ref, w_ref, b_ref, y_ref, st_ref, acc_ref):
    y = _dot_t(x_ref[...], w_ref[...]) + b_ref[...]
    y_ref[...] = y
    i = pl.program_id(0)

    @pl.when(i == 0)
    def _():
        acc_ref[...] = jnp.zeros_like(acc_ref)

    acc_ref[0, :] += jnp.sum(y, axis=0)
    acc_ref[1, :] += jnp.sum(y * y, axis=0)

    @pl.when(i == nsteps - 1)
    def _():
        st_ref[...] = acc_ref[...]


def _encode(x, w, b):
    n, raw = x.shape
    blk = _rowblk(n)
    nsteps = n // blk
    y, st = pl.pallas_call(
        functools.partial(_enc_body, nsteps),
        grid=(nsteps,),
        in_specs=[
            pl.BlockSpec((blk, raw), lambda i: (i, 0)),
            pl.BlockSpec((H, raw), lambda i: (0, 0)),
            pl.BlockSpec((1, H), lambda i: (0, 0)),
        ],
        out_specs=[
            pl.BlockSpec((blk, H), lambda i: (i, 0)),
            pl.BlockSpec((8, H), lambda i: (0, 0)),
        ],
        out_shape=[
            jax.ShapeDtypeStruct((n, H), jnp.float32),
            jax.ShapeDtypeStruct((8, H), jnp.float32),
        ],
        scratch_shapes=[pltpu.VMEM((8, H), jnp.float32)],
    )(x, w, b.reshape(1, H))
    return y, st


def _bn_body(n, relu, x_ref, st_ref, g_ref, be_ref, o_ref):
    mu = st_ref[0, :] / n
    var = st_ref[1, :] / n - mu * mu
    scale = jax.lax.rsqrt(var + _EPS) * g_ref[0, :]
    y = (x_ref[...] - mu[None, :]) * scale[None, :] + be_ref[...]
    if relu:
        y = jnp.maximum(y, 0.0)
    o_ref[...] = y


def _bn_apply(x, st, g, be, relu):
    n = x.shape[0]
    blk = _rowblk(n)
    return pl.pallas_call(
        functools.partial(_bn_body, float(n), relu),
        grid=(n // blk,),
        in_specs=[
            pl.BlockSpec((blk, H), lambda i: (i, 0)),
            pl.BlockSpec((8, H), lambda i: (0, 0)),
            pl.BlockSpec((1, H), lambda i: (0, 0)),
            pl.BlockSpec((1, H), lambda i: (0, 0)),
        ],
        out_specs=pl.BlockSpec((blk, H), lambda i: (i, 0)),
        out_shape=jax.ShapeDtypeStruct((n, H), jnp.float32),
    )(x, st, g.reshape(1, H), be.reshape(1, H))


def _combine_body(k, nsteps, want_stats, *refs):
    sums = refs[0:k]
    cnts = refs[k:2 * k]
    hd_ref = refs[2 * k]
    wls = refs[2 * k + 1:3 * k + 1]
    bls = refs[3 * k + 1:4 * k + 1]
    wrs = refs[4 * k + 1:5 * k + 1]
    y_ref = refs[5 * k + 1]
    hd = hd_ref[...]
    y = None
    for j in range(k):
        c = cnts[j][...]
        cnt = (c[0] + c[1])[:, 0:1]
        inv = 1.0 / jnp.maximum(cnt, 1.0)
        s = sums[j][...]
        mean = (s[0] + s[1]) * inv
        r = _dot_t(mean, wls[j][...]) + bls[j][...] + _dot_t(hd, wrs[j][...])
        y = r if y is None else jnp.maximum(y, r)
    y_ref[...] = y
    if want_stats:
        st_ref, acc_ref = refs[5 * k + 2], refs[5 * k + 3]
        i = pl.program_id(0)

        @pl.when(i == 0)
        def _():
            acc_ref[...] = jnp.zeros_like(acc_ref)

        acc_ref[0, :] += jnp.sum(y, axis=0)
        acc_ref[1, :] += jnp.sum(y * y, axis=0)

        @pl.when(i == nsteps - 1)
        def _():
            st_ref[...] = acc_ref[...]


def _combine(sums, cnts, hd, wls, bls, wrs, want_stats):
    k = len(sums)
    n = hd.shape[0]
    blk = _rowblk(n)
    if blk == n and sums[0].shape[1] != n:
        sums = [a[:, :n] for a in sums]
        cnts = [a[:, :n] for a in cnts]
    nsteps = n // blk
    in_specs = (
        [pl.BlockSpec((2, blk, H), lambda i: (0, i, 0))] * k
        + [pl.BlockSpec((2, blk, H), lambda i: (0, i, 0))] * k
        + [pl.BlockSpec((blk, H), lambda i: (i, 0))]
        + [pl.BlockSpec((H, H), lambda i: (0, 0))] * k
        + [pl.BlockSpec((1, H), lambda i: (0, 0))] * k
        + [pl.BlockSpec((H, H), lambda i: (0, 0))] * k
    )
    out_specs = [pl.BlockSpec((blk, H), lambda i: (i, 0))]
    out_shape = [jax.ShapeDtypeStruct((n, H), jnp.float32)]
    scratch = []
    if want_stats:
        out_specs.append(pl.BlockSpec((8, H), lambda i: (0, 0)))
        out_shape.append(jax.ShapeDtypeStruct((8, H), jnp.float32))
        scratch.append(pltpu.VMEM((8, H), jnp.float32))
    args = list(sums) + list(cnts) + [hd] + list(wls) \
        + [b.reshape(1, H) for b in bls] + list(wrs)
    res = pl.pallas_call(
        functools.partial(_combine_body, k, nsteps, want_stats),
        grid=(nsteps,),
        in_specs=in_specs,
        out_specs=out_specs,
        out_shape=out_shape,
        scratch_shapes=scratch,
    )(*args)
    return res if want_stats else (res[0], None)


def _final_body(x_ref, w_ref, b_ref, o_ref):
    x = jnp.maximum(x_ref[...], 0.0)
    o_ref[...] = _dot_t(x, w_ref[...]) + b_ref[...]


def _final(x, w, b):
    n = x.shape[0]
    blk = _rowblk(n)
    return pl.pallas_call(
        _final_body,
        grid=(n // blk,),
        in_specs=[
            pl.BlockSpec((blk, H), lambda i: (i, 0)),
            pl.BlockSpec((OUT, H), lambda i: (0, 0)),
            pl.BlockSpec((1, OUT), lambda i: (0, 0)),
        ],
        out_specs=pl.BlockSpec((blk, OUT), lambda i: (i, 0)),
        out_shape=jax.ShapeDtypeStruct((n, OUT), jnp.float32),
    )(x, w, b.reshape(1, OUT))


_NALIGN = {t: max(2048, -(-(N_NODES[t] + 1) // 2048) * 2048) for t in NODE_TYPES}


def _epad(e):
    return -(-e // 8192) * 8192


def _make_sc_agg(e_pad, n_align, with_cnt):
    nsteps = e_pad // 32 // 128
    rows_pt = n_align // 16
    nzero = rows_pt // 128
    mesh = plsc.VectorSubcoreMesh(core_axis_name="c", subcore_axis_name="s",
                                  num_cores=2, num_subcores=16)
    out_type = [jax.ShapeDtypeStruct((2, n_align, 128), jnp.float32)]
    if with_cnt:
        out_type.append(jax.ShapeDtypeStruct((2, n_align, 128), jnp.float32))
    scratch = [
        pltpu.VMEM_SHARED((n_align, 32), jnp.float32),
        pltpu.VMEM((nsteps, 128), jnp.int32),
        pltpu.VMEM((nsteps, 128), jnp.int32),
        pltpu.VMEM((128, 32), jnp.float32),
        pltpu.VMEM((128, 32), jnp.float32),
        pltpu.SemaphoreType.DMA,
    ]

    def body(*refs):
        if with_cnt:
            h4, srcq, dst, out_s, out_c, acc, src_v, dst_v, rows_v, zbuf, sem = refs
        else:
            h4, srcq, dst, out_s, acc, src_v, dst_v, rows_v, zbuf, sem = refs
            out_c = None
        cid = lax.axis_index("c")
        sid = lax.axis_index("s")
        wid = cid * 16 + sid
        r0 = sid * rows_pt
        z16 = jnp.zeros((16,), jnp.float32)

        def zb(i, c):
            zbuf[i, 0:16] = z16
            zbuf[i, 16:32] = z16
            return c
        lax.fori_loop(0, 128, zb, 0)
        pltpu.sync_copy(dst.at[wid], dst_v)
        npass = 5 if with_cnt else 4
        for p in range(npass):
            is_cnt = p == 4

            def zl(i, c):
                pltpu.sync_copy(zbuf, acc.at[pl.ds(r0 + i * 128, 128)])
                return c
            lax.fori_loop(0, nzero, zl, 0)
            if is_cnt:
                o16 = jnp.ones((16,), jnp.float32)

                def ob(i, c):
                    rows_v[i, 0:16] = o16
                    rows_v[i, 16:32] = o16
                    return c
                lax.fori_loop(0, 128, ob, 0)
            else:
                pltpu.sync_copy(srcq.at[p, wid], src_v)
            plsc.subcore_barrier()

            if is_cnt:
                def step_c(j, c):
                    pltpu.sync_copy(rows_v, acc.at[dst_v.at[j]], add=True)
                    return c
                lax.fori_loop(0, nsteps, step_c, 0)
            else:
                def step(j, c):
                    pltpu.async_copy(h4.at[src_v.at[j]], rows_v, sem).wait()
                    pltpu.sync_copy(rows_v, acc.at[dst_v.at[j]], add=True)
                    return c
                lax.fori_loop(0, nsteps, step, 0)
            plsc.subcore_barrier()
            if is_cnt:
                pltpu.sync_copy(acc.at[pl.ds(r0, rows_pt)],
                                out_c.at[cid, pl.ds(r0, rows_pt), pl.ds(0, 32)])
            else:
                pltpu.sync_copy(acc.at[pl.ds(r0, rows_pt)],
                                out_s.at[cid, pl.ds(r0, rows_pt),
                                         pl.ds(32 * p, 32)])

    return pl.kernel(
        body, out_type=out_type, mesh=mesh, scratch_types=scratch,
        compiler_params=pltpu.CompilerParams(use_tc_tiling_on_sc=False))


def _prep_edges(ei, n_dst, e_pad):
    e = ei.shape[1]
    nsteps = e_pad // 32 // 128
    srcq = (ei[0] * 4)[None, :] + jnp.arange(4, dtype=jnp.int32)[:, None]
    srcq = jnp.pad(srcq, ((0, 0), (0, e_pad - e)))
    dst = jnp.pad(ei[1], (0, e_pad - e), constant_values=n_dst)
    return (srcq.reshape(4, 32, nsteps, 128), dst.reshape(32, nsteps, 128))


def _segment_sums(h, prepped, with_cnt):
    sums, cnts = {}, {}
    for s, d, e in EDGE_TYPES:
        srcq, dst = prepped[(s, d)]
        h4 = h[s].reshape(4 * N_NODES[s], 32)
        fn = _make_sc_agg(_epad(e), _NALIGN[d], with_cnt)
        res = fn(h4, srcq, dst)
        if with_cnt:
            sums[(s, d)], cnts[(s, d)] = res
        else:
            sums[(s, d)] = res[0] if isinstance(res, (tuple, list)) else res
    return sums, cnts


def _hetero_layer(h, convp, prepped, cnts, want_stats, with_cnt=False):
    sums, new_cnts = _segment_sums(h, prepped, with_cnt)
    if with_cnt:
        cnts = new_cnts
    incoming = {}
    for s, d, _ in EDGE_TYPES:
        incoming.setdefault(d, []).append((s, d))
    out = {}
    for d, keys in incoming.items():
        p = [convp[s + '__' + dd] for s, dd in keys]
        y, st = _combine([sums[k] for k in keys], [cnts[k] for k in keys],
                         h[d], [q['wl'] for q in p], [q['bl'] for q in p],
                         [q['wr'] for q in p], want_stats)
        out[d] = (y, st)
    return out, cnts


def kernel(x_Patient, x_Admission, params, ei_Patient_Admission, ei_Admission_Patient, ei_Admission_Diagnosis, ei_Diagnosis_Admission, ei_Admission_Medication, ei_Medication_Admission, ei_Admission_Procedure, ei_Procedure_Admission, ei_Admission_LabTest, ei_LabTest_Admission):
    eis = {
        ('Patient', 'Admission'): ei_Patient_Admission,
        ('Admission', 'Patient'): ei_Admission_Patient,
        ('Admission', 'Diagnosis'): ei_Admission_Diagnosis,
        ('Diagnosis', 'Admission'): ei_Diagnosis_Admission,
        ('Admission', 'Medication'): ei_Admission_Medication,
        ('Medication', 'Admission'): ei_Medication_Admission,
        ('Admission', 'Procedure'): ei_Admission_Procedure,
        ('Procedure', 'Admission'): ei_Procedure_Admission,
        ('Admission', 'LabTest'): ei_Admission_LabTest,
        ('LabTest', 'Admission'): ei_LabTest_Admission,
    }
    enc = params['enc']
    h = {}
    yp, stp = _encode(x_Patient, enc['pat_w'], enc['pat_b'])
    h['Patient'] = _bn_apply(yp, stp, enc['pat_g'], enc['pat_be'], relu=False)
    ya, sta = _encode(x_Admission, enc['adm_w'], enc['adm_b'])
    h['Admission'] = _bn_apply(ya, sta, enc['adm_g'], enc['adm_be'], relu=False)
    for t in EMB_TYPES:
        h[t] = enc['emb_' + t]

    prepped = {(s, d): _prep_edges(eis[(s, d)], N_NODES[d], _epad(e))
               for s, d, e in EDGE_TYPES}

    cnts = None
    for L, bn in (('conv1', 'bn1'), ('conv2', 'bn2')):
        res, cnts = _hetero_layer(h, params[L], prepped, cnts,
                                  want_stats=True, with_cnt=(L == 'conv1'))
        h = {t: _bn_apply(res[t][0], res[t][1], params[bn][t]['g'],
                          params[bn][t]['b'], relu=True)
             for t in NODE_TYPES}

    res, _ = _hetero_layer(h, params['conv3'], prepped, cnts, want_stats=False)
    out = [_final(res[t][0], params['lin'][t]['w'], params['lin'][t]['b'])
           for t in NODE_TYPES]
    return jnp.concatenate(out, axis=0)


# speedup vs baseline: 1.2640x; 1.0978x over previous
import functools

import jax
import jax.numpy as jnp
from jax import lax
from jax.experimental import pallas as pl
from jax.experimental.pallas import tpu as pltpu
from jax.experimental.pallas import tpu_sc as plsc

H = 128
OUT = 64
NODE_TYPES = ['Patient', 'Admission', 'Diagnosis', 'Medication', 'Procedure', 'LabTest']
N_NODES = {'Patient': 20000, 'Admission': 40000, 'Diagnosis': 2000, 'Medication': 1000, 'Procedure': 2000, 'LabTest': 500}
EMB_TYPES = ['Diagnosis', 'Medication', 'Procedure', 'LabTest']
EDGE_TYPES = [('Patient', 'Admission', 40000), ('Admission', 'Patient', 40000), ('Admission', 'Diagnosis', 80000), ('Diagnosis', 'Admission', 80000), ('Admission', 'Medication', 60000), ('Medication', 'Admission', 60000), ('Admission', 'Procedure', 40000), ('Procedure', 'Admission', 40000), ('Admission', 'LabTest', 80000), ('LabTest', 'Admission', 80000)]

_EPS = 1e-5


def _rowblk(n):
    if n % 2000 == 0:
        return 2000
    if n % 1000 == 0:
        return 1000
    return n


def _dot_t(x, w):
    return jax.lax.dot_general(x, w, (((1,), (1,)), ((), ())),
                               preferred_element_type=jnp.float32)


def _enc_body(nsteps, x_ref, w_ref, b_ref, y_ref, st_ref, acc_ref):
    y = _dot_t(x_ref[...], w_ref[...]) + b_ref[...]
    y_ref[...] = y
    i = pl.program_id(0)

    @pl.when(i == 0)
    def _():
        acc_ref[...] = jnp.zeros_like(acc_ref)

    acc_ref[0, :] += jnp.sum(y, axis=0)
    acc_ref[1, :] += jnp.sum(y * y, axis=0)

    @pl.when(i == nsteps - 1)
    def _():
        st_ref[...] = acc_ref[...]


def _encode(x, w, b):
    n, raw = x.shape
    blk = _rowblk(n)
    nsteps = n // blk
    y, st = pl.pallas_call(
        functools.partial(_enc_body, nsteps),
        grid=(nsteps,),
        in_specs=[
            pl.BlockSpec((blk, raw), lambda i: (i, 0)),
            pl.BlockSpec((H, raw), lambda i: (0, 0)),
            pl.BlockSpec((1, H), lambda i: (0, 0)),
        ],
        out_specs=[
            pl.BlockSpec((blk, H), lambda i: (i, 0)),
            pl.BlockSpec((8, H), lambda i: (0, 0)),
        ],
        out_shape=[
            jax.ShapeDtypeStruct((n, H), jnp.float32),
            jax.ShapeDtypeStruct((8, H), jnp.float32),
        ],
        scratch_shapes=[pltpu.VMEM((8, H), jnp.float32)],
    )(x, w, b.reshape(1, H))
    return y, st


def _bn_body(n, relu, x_ref, st_ref, g_ref, be_ref, o_ref):
    mu = st_ref[0, :] / n
    var = st_ref[1, :] / n - mu * mu
    scale = jax.lax.rsqrt(var + _EPS) * g_ref[0, :]
    y = (x_ref[...] - mu[None, :]) * scale[None, :] + be_ref[...]
    if relu:
        y = jnp.maximum(y, 0.0)
    o_ref[...] = y


def _bn_apply(x, st, g, be, relu):
    n = x.shape[0]
    blk = _rowblk(n)
    return pl.pallas_call(
        functools.partial(_bn_body, float(n), relu),
        grid=(n // blk,),
        in_specs=[
            pl.BlockSpec((blk, H), lambda i: (i, 0)),
            pl.BlockSpec((8, H), lambda i: (0, 0)),
            pl.BlockSpec((1, H), lambda i: (0, 0)),
            pl.BlockSpec((1, H), lambda i: (0, 0)),
        ],
        out_specs=pl.BlockSpec((blk, H), lambda i: (i, 0)),
        out_shape=jax.ShapeDtypeStruct((n, H), jnp.float32),
    )(x, st, g.reshape(1, H), be.reshape(1, H))


def _combine_body(k, nsteps, want_stats, *refs):
    sums = refs[0:k]
    cnts = refs[k:2 * k]
    hd_ref = refs[2 * k]
    wls = refs[2 * k + 1:3 * k + 1]
    bls = refs[3 * k + 1:4 * k + 1]
    wrs = refs[4 * k + 1:5 * k + 1]
    y_ref = refs[5 * k + 1]
    hd = hd_ref[...]
    y = None
    for j in range(k):
        c = cnts[j][...]
        cnt = (c[0] + c[1])[:, 0:1]
        inv = 1.0 / jnp.maximum(cnt, 1.0)
        s = sums[j][...]
        mean = (s[0] + s[1]) * inv
        r = _dot_t(mean, wls[j][...]) + bls[j][...] + _dot_t(hd, wrs[j][...])
        y = r if y is None else jnp.maximum(y, r)
    y_ref[...] = y
    if want_stats:
        st_ref, acc_ref = refs[5 * k + 2], refs[5 * k + 3]
        i = pl.program_id(0)

        @pl.when(i == 0)
        def _():
            acc_ref[...] = jnp.zeros_like(acc_ref)

        acc_ref[0, :] += jnp.sum(y, axis=0)
        acc_ref[1, :] += jnp.sum(y * y, axis=0)

        @pl.when(i == nsteps - 1)
        def _():
            st_ref[...] = acc_ref[...]


def _combine(sums, cnts, hd, wls, bls, wrs, want_stats):
    k = len(sums)
    n = hd.shape[0]
    blk = _rowblk(n)
    if blk == n and sums[0].shape[1] != n:
        sums = [a[:, :n] for a in sums]
        cnts = [a[:, :n] for a in cnts]
    nsteps = n // blk
    in_specs = (
        [pl.BlockSpec((2, blk, H), lambda i: (0, i, 0))] * k
        + [pl.BlockSpec((2, blk, H), lambda i: (0, i, 0))] * k
        + [pl.BlockSpec((blk, H), lambda i: (i, 0))]
        + [pl.BlockSpec((H, H), lambda i: (0, 0))] * k
        + [pl.BlockSpec((1, H), lambda i: (0, 0))] * k
        + [pl.BlockSpec((H, H), lambda i: (0, 0))] * k
    )
    out_specs = [pl.BlockSpec((blk, H), lambda i: (i, 0))]
    out_shape = [jax.ShapeDtypeStruct((n, H), jnp.float32)]
    scratch = []
    if want_stats:
        out_specs.append(pl.BlockSpec((8, H), lambda i: (0, 0)))
        out_shape.append(jax.ShapeDtypeStruct((8, H), jnp.float32))
        scratch.append(pltpu.VMEM((8, H), jnp.float32))
    args = list(sums) + list(cnts) + [hd] + list(wls) \
        + [b.reshape(1, H) for b in bls] + list(wrs)
    res = pl.pallas_call(
        functools.partial(_combine_body, k, nsteps, want_stats),
        grid=(nsteps,),
        in_specs=in_specs,
        out_specs=out_specs,
        out_shape=out_shape,
        scratch_shapes=scratch,
    )(*args)
    return res if want_stats else (res[0], None)


def _final_body(x_ref, w_ref, b_ref, o_ref):
    x = jnp.maximum(x_ref[...], 0.0)
    o_ref[...] = _dot_t(x, w_ref[...]) + b_ref[...]


def _final(x, w, b):
    n = x.shape[0]
    blk = _rowblk(n)
    return pl.pallas_call(
        _final_body,
        grid=(n // blk,),
        in_specs=[
            pl.BlockSpec((blk, H), lambda i: (i, 0)),
            pl.BlockSpec((OUT, H), lambda i: (0, 0)),
            pl.BlockSpec((1, OUT), lambda i: (0, 0)),
        ],
        out_specs=pl.BlockSpec((blk, OUT), lambda i: (i, 0)),
        out_shape=jax.ShapeDtypeStruct((n, OUT), jnp.float32),
    )(x, w, b.reshape(1, OUT))


_NALIGN = {t: max(2048, -(-(N_NODES[t] + 1) // 2048) * 2048) for t in NODE_TYPES}


def _epad(e):
    return -(-e // 8192) * 8192


def _make_sc_agg(e_pad, n_align, with_cnt):
    nsteps = e_pad // 32 // 128
    rows_pt = n_align // 16
    nzero = rows_pt // 128
    mesh = plsc.VectorSubcoreMesh(core_axis_name="c", subcore_axis_name="s",
                                  num_cores=2, num_subcores=16)
    out_type = [jax.ShapeDtypeStruct((2, n_align, 128), jnp.float32)]
    if with_cnt:
        out_type.append(jax.ShapeDtypeStruct((2, n_align, 128), jnp.float32))
    scratch = [
        pltpu.VMEM_SHARED((n_align, 32), jnp.float32),
        pltpu.VMEM((nsteps, 128), jnp.int32),
        pltpu.VMEM((nsteps, 128), jnp.int32),
        pltpu.VMEM((8, 128, 32), jnp.float32),
        pltpu.VMEM((128, 32), jnp.float32),
        pltpu.SemaphoreType.DMA,
        pltpu.SemaphoreType.DMA,
    ]

    def body(*refs):
        if with_cnt:
            (h4, srcq, dst, out_s, out_c,
             acc, src_v, dst_v, rows_v, zbuf, sem, sem2) = refs
        else:
            (h4, srcq, dst, out_s,
             acc, src_v, dst_v, rows_v, zbuf, sem, sem2) = refs
            out_c = None
        cid = lax.axis_index("c")
        sid = lax.axis_index("s")
        wid = cid * 16 + sid
        r0 = sid * rows_pt
        z16 = jnp.zeros((16,), jnp.float32)

        def zb(i, c):
            zbuf[i, 0:16] = z16
            zbuf[i, 16:32] = z16
            return c
        lax.fori_loop(0, 128, zb, 0)
        pltpu.sync_copy(dst.at[wid], dst_v)
        npass = 5 if with_cnt else 4
        for p in range(npass):
            is_cnt = p == 4

            def zl(i, c):
                pltpu.sync_copy(zbuf, acc.at[pl.ds(r0 + i * 128, 128)])
                return c
            lax.fori_loop(0, nzero, zl, 0)
            if is_cnt:
                o16 = jnp.ones((16,), jnp.float32)

                def ob(i, c):
                    rows_v[0, i, 0:16] = o16
                    rows_v[0, i, 16:32] = o16
                    return c
                lax.fori_loop(0, 128, ob, 0)
            else:
                pltpu.sync_copy(srcq.at[p, wid], src_v)
            plsc.subcore_barrier()

            if is_cnt:
                def fire_c(j, c):
                    pltpu.async_copy(rows_v.at[0], acc.at[dst_v.at[j]], sem2,
                                     add=True)
                    return c
                lax.fori_loop(0, nsteps, fire_c, 0)

                def drain_c(j, c):
                    pltpu.make_async_copy(rows_v.at[0], acc.at[dst_v.at[j]],
                                          sem2).wait()
                    return c
                lax.fori_loop(0, nsteps, drain_c, 0)
            else:
                def chunk(base, nb):
                    for b in range(nb):
                        pltpu.async_copy(h4.at[src_v.at[base + b]],
                                         rows_v.at[b], sem)
                    for b in range(nb):
                        pltpu.make_async_copy(h4.at[src_v.at[base + b]],
                                              rows_v.at[b], sem).wait()
                        pltpu.async_copy(rows_v.at[b],
                                         acc.at[dst_v.at[base + b]], sem2,
                                         add=True)
                    for b in range(nb):
                        pltpu.make_async_copy(rows_v.at[b],
                                              acc.at[dst_v.at[base + b]],
                                              sem2).wait()

                nchunks, tail = divmod(nsteps, 8)

                def chunk_loop(c, x):
                    chunk(c * 8, 8)
                    return x
                lax.fori_loop(0, nchunks, chunk_loop, 0)
                if tail:
                    chunk(nchunks * 8, tail)
            plsc.subcore_barrier()
            if is_cnt:
                pltpu.sync_copy(acc.at[pl.ds(r0, rows_pt)],
                                out_c.at[cid, pl.ds(r0, rows_pt), pl.ds(0, 32)])
            else:
                pltpu.sync_copy(acc.at[pl.ds(r0, rows_pt)],
                                out_s.at[cid, pl.ds(r0, rows_pt),
                                         pl.ds(32 * p, 32)])

    return pl.kernel(
        body, out_type=out_type, mesh=mesh, scratch_types=scratch,
        compiler_params=pltpu.CompilerParams(use_tc_tiling_on_sc=False))


def _prep_edges(ei, n_dst, e_pad):
    e = ei.shape[1]
    nsteps = e_pad // 32 // 128
    srcq = (ei[0] * 4)[None, :] + jnp.arange(4, dtype=jnp.int32)[:, None]
    srcq = jnp.pad(srcq, ((0, 0), (0, e_pad - e)))
    dst = jnp.pad(ei[1], (0, e_pad - e), constant_values=n_dst)
    return (srcq.reshape(4, 32, nsteps, 128), dst.reshape(32, nsteps, 128))


def _segment_sums(h, prepped, with_cnt):
    sums, cnts = {}, {}
    for s, d, e in EDGE_TYPES:
        srcq, dst = prepped[(s, d)]
        h4 = h[s].reshape(4 * N_NODES[s], 32)
        fn = _make_sc_agg(_epad(e), _NALIGN[d], with_cnt)
        res = fn(h4, srcq, dst)
        if with_cnt:
            sums[(s, d)], cnts[(s, d)] = res
        else:
            sums[(s, d)] = res[0] if isinstance(res, (tuple, list)) else res
    return sums, cnts


def _hetero_layer(h, convp, prepped, cnts, want_stats, with_cnt=False):
    sums, new_cnts = _segment_sums(h, prepped, with_cnt)
    if with_cnt:
        cnts = new_cnts
    incoming = {}
    for s, d, _ in EDGE_TYPES:
        incoming.setdefault(d, []).append((s, d))
    out = {}
    for d, keys in incoming.items():
        p = [convp[s + '__' + dd] for s, dd in keys]
        y, st = _combine([sums[k] for k in keys], [cnts[k] for k in keys],
                         h[d], [q['wl'] for q in p], [q['bl'] for q in p],
                         [q['wr'] for q in p], want_stats)
        out[d] = (y, st)
    return out, cnts


def kernel(x_Patient, x_Admission, params, ei_Patient_Admission, ei_Admission_Patient, ei_Admission_Diagnosis, ei_Diagnosis_Admission, ei_Admission_Medication, ei_Medication_Admission, ei_Admission_Procedure, ei_Procedure_Admission, ei_Admission_LabTest, ei_LabTest_Admission):
    eis = {
        ('Patient', 'Admission'): ei_Patient_Admission,
        ('Admission', 'Patient'): ei_Admission_Patient,
        ('Admission', 'Diagnosis'): ei_Admission_Diagnosis,
        ('Diagnosis', 'Admission'): ei_Diagnosis_Admission,
        ('Admission', 'Medication'): ei_Admission_Medication,
        ('Medication', 'Admission'): ei_Medication_Admission,
        ('Admission', 'Procedure'): ei_Admission_Procedure,
        ('Procedure', 'Admission'): ei_Procedure_Admission,
        ('Admission', 'LabTest'): ei_Admission_LabTest,
        ('LabTest', 'Admission'): ei_LabTest_Admission,
    }
    enc = params['enc']
    h = {}
    yp, stp = _encode(x_Patient, enc['pat_w'], enc['pat_b'])
    h['Patient'] = _bn_apply(yp, stp, enc['pat_g'], enc['pat_be'], relu=False)
    ya, sta = _encode(x_Admission, enc['adm_w'], enc['adm_b'])
    h['Admission'] = _bn_apply(ya, sta, enc['adm_g'], enc['adm_be'], relu=False)
    for t in EMB_TYPES:
        h[t] = enc['emb_' + t]

    prepped = {(s, d): _prep_edges(eis[(s, d)], N_NODES[d], _epad(e))
               for s, d, e in EDGE_TYPES}

    cnts = None
    for L, bn in (('conv1', 'bn1'), ('conv2', 'bn2')):
        res, cnts = _hetero_layer(h, params[L], prepped, cnts,
                                  want_stats=True, with_cnt=(L == 'conv1'))
        h = {t: _bn_apply(res[t][0], res[t][1], params[bn][t]['g'],
                          params[bn][t]['b'], relu=True)
             for t in NODE_TYPES}

    res, _ = _hetero_layer(h, params['conv3'], prepped, cnts, want_stats=False)
    out = [_final(res[t][0], params['lin'][t]['w'], params['lin'][t]['b'])
           for t in NODE_TYPES]
    return jnp.concatenate(out, axis=0)


# speedup vs baseline: 1.2777x; 1.0109x over previous
import functools

import jax
import jax.numpy as jnp
from jax import lax
from jax.experimental import pallas as pl
from jax.experimental.pallas import tpu as pltpu
from jax.experimental.pallas import tpu_sc as plsc

H = 128
OUT = 64
NODE_TYPES = ['Patient', 'Admission', 'Diagnosis', 'Medication', 'Procedure', 'LabTest']
N_NODES = {'Patient': 20000, 'Admission': 40000, 'Diagnosis': 2000, 'Medication': 1000, 'Procedure': 2000, 'LabTest': 500}
EMB_TYPES = ['Diagnosis', 'Medication', 'Procedure', 'LabTest']
EDGE_TYPES = [('Patient', 'Admission', 40000), ('Admission', 'Patient', 40000), ('Admission', 'Diagnosis', 80000), ('Diagnosis', 'Admission', 80000), ('Admission', 'Medication', 60000), ('Medication', 'Admission', 60000), ('Admission', 'Procedure', 40000), ('Procedure', 'Admission', 40000), ('Admission', 'LabTest', 80000), ('LabTest', 'Admission', 80000)]

_EPS = 1e-5


def _rowblk(n):
    if n % 2000 == 0:
        return 2000
    if n % 1000 == 0:
        return 1000
    return n


def _dot_t(x, w):
    return jax.lax.dot_general(x, w, (((1,), (1,)), ((), ())),
                               preferred_element_type=jnp.float32)


def _enc_body(nsteps, x_ref, w_ref, b_ref, y_ref, st_ref, acc_ref):
    y = _dot_t(x_ref[...], w_ref[...]) + b_ref[...]
    y_ref[...] = y
    i = pl.program_id(0)

    @pl.when(i == 0)
    def _():
        acc_ref[...] = jnp.zeros_like(acc_ref)

    acc_ref[0, :] += jnp.sum(y, axis=0)
    acc_ref[1, :] += jnp.sum(y * y, axis=0)

    @pl.when(i == nsteps - 1)
    def _():
        st_ref[...] = acc_ref[...]


def _encode(x, w, b):
    n, raw = x.shape
    blk = _rowblk(n)
    nsteps = n // blk
    y, st = pl.pallas_call(
        functools.partial(_enc_body, nsteps),
        grid=(nsteps,),
        in_specs=[
            pl.BlockSpec((blk, raw), lambda i: (i, 0)),
            pl.BlockSpec((H, raw), lambda i: (0, 0)),
            pl.BlockSpec((1, H), lambda i: (0, 0)),
        ],
        out_specs=[
            pl.BlockSpec((blk, H), lambda i: (i, 0)),
            pl.BlockSpec((8, H), lambda i: (0, 0)),
        ],
        out_shape=[
            jax.ShapeDtypeStruct((n, H), jnp.float32),
            jax.ShapeDtypeStruct((8, H), jnp.float32),
        ],
        scratch_shapes=[pltpu.VMEM((8, H), jnp.float32)],
    )(x, w, b.reshape(1, H))
    return y, st


def _bn_body(n, relu, x_ref, st_ref, g_ref, be_ref, o_ref):
    mu = st_ref[0, :] / n
    var = st_ref[1, :] / n - mu * mu
    scale = jax.lax.rsqrt(var + _EPS) * g_ref[0, :]
    y = (x_ref[...] - mu[None, :]) * scale[None, :] + be_ref[...]
    if relu:
        y = jnp.maximum(y, 0.0)
    o_ref[...] = y


def _bn_apply(x, st, g, be, relu):
    n = x.shape[0]
    blk = _rowblk(n)
    return pl.pallas_call(
        functools.partial(_bn_body, float(n), relu),
        grid=(n // blk,),
        in_specs=[
            pl.BlockSpec((blk, H), lambda i: (i, 0)),
            pl.BlockSpec((8, H), lambda i: (0, 0)),
            pl.BlockSpec((1, H), lambda i: (0, 0)),
            pl.BlockSpec((1, H), lambda i: (0, 0)),
        ],
        out_specs=pl.BlockSpec((blk, H), lambda i: (i, 0)),
        out_shape=jax.ShapeDtypeStruct((n, H), jnp.float32),
    )(x, st, g.reshape(1, H), be.reshape(1, H))


def _combine_body(k, nsteps, want_stats, *refs):
    sums = refs[0:k]
    cnts = refs[k:2 * k]
    hd_ref = refs[2 * k]
    wls = refs[2 * k + 1:3 * k + 1]
    bls = refs[3 * k + 1:4 * k + 1]
    wrs = refs[4 * k + 1:5 * k + 1]
    y_ref = refs[5 * k + 1]
    hd = hd_ref[...]
    y = None
    for j in range(k):
        c = cnts[j][...]
        cnt = (c[0] + c[1])[:, 0:1]
        inv = 1.0 / jnp.maximum(cnt, 1.0)
        s = sums[j][...]
        mean = (s[0] + s[1]) * inv
        r = _dot_t(mean, wls[j][...]) + bls[j][...] + _dot_t(hd, wrs[j][...])
        y = r if y is None else jnp.maximum(y, r)
    y_ref[...] = y
    if want_stats:
        st_ref, acc_ref = refs[5 * k + 2], refs[5 * k + 3]
        i = pl.program_id(0)

        @pl.when(i == 0)
        def _():
            acc_ref[...] = jnp.zeros_like(acc_ref)

        acc_ref[0, :] += jnp.sum(y, axis=0)
        acc_ref[1, :] += jnp.sum(y * y, axis=0)

        @pl.when(i == nsteps - 1)
        def _():
            st_ref[...] = acc_ref[...]


def _combine(sums, cnts, hd, wls, bls, wrs, want_stats):
    k = len(sums)
    n = hd.shape[0]
    blk = _rowblk(n)
    if blk == n and sums[0].shape[1] != n:
        sums = [a[:, :n] for a in sums]
        cnts = [a[:, :n] for a in cnts]
    nsteps = n // blk
    in_specs = (
        [pl.BlockSpec((2, blk, H), lambda i: (0, i, 0))] * k
        + [pl.BlockSpec((2, blk, H), lambda i: (0, i, 0))] * k
        + [pl.BlockSpec((blk, H), lambda i: (i, 0))]
        + [pl.BlockSpec((H, H), lambda i: (0, 0))] * k
        + [pl.BlockSpec((1, H), lambda i: (0, 0))] * k
        + [pl.BlockSpec((H, H), lambda i: (0, 0))] * k
    )
    out_specs = [pl.BlockSpec((blk, H), lambda i: (i, 0))]
    out_shape = [jax.ShapeDtypeStruct((n, H), jnp.float32)]
    scratch = []
    if want_stats:
        out_specs.append(pl.BlockSpec((8, H), lambda i: (0, 0)))
        out_shape.append(jax.ShapeDtypeStruct((8, H), jnp.float32))
        scratch.append(pltpu.VMEM((8, H), jnp.float32))
    args = list(sums) + list(cnts) + [hd] + list(wls) \
        + [b.reshape(1, H) for b in bls] + list(wrs)
    res = pl.pallas_call(
        functools.partial(_combine_body, k, nsteps, want_stats),
        grid=(nsteps,),
        in_specs=in_specs,
        out_specs=out_specs,
        out_shape=out_shape,
        scratch_shapes=scratch,
    )(*args)
    return res if want_stats else (res[0], None)


def _final_body(x_ref, w_ref, b_ref, o_ref):
    x = jnp.maximum(x_ref[...], 0.0)
    o_ref[...] = _dot_t(x, w_ref[...]) + b_ref[...]


def _final(x, w, b):
    n = x.shape[0]
    blk = _rowblk(n)
    return pl.pallas_call(
        _final_body,
        grid=(n // blk,),
        in_specs=[
            pl.BlockSpec((blk, H), lambda i: (i, 0)),
            pl.BlockSpec((OUT, H), lambda i: (0, 0)),
            pl.BlockSpec((1, OUT), lambda i: (0, 0)),
        ],
        out_specs=pl.BlockSpec((blk, OUT), lambda i: (i, 0)),
        out_shape=jax.ShapeDtypeStruct((n, OUT), jnp.float32),
    )(x, w, b.reshape(1, OUT))


_NALIGN = {t: max(2048, -(-(N_NODES[t] + 1) // 2048) * 2048) for t in NODE_TYPES}


def _epad(e):
    return -(-e // 8192) * 8192


def _make_sc_agg(e_pad, n_align, with_cnt):
    nsteps = e_pad // 32 // 128
    rows_pt = n_align // 16
    nzero = rows_pt // 128
    mesh = plsc.VectorSubcoreMesh(core_axis_name="c", subcore_axis_name="s",
                                  num_cores=2, num_subcores=16)
    out_type = [jax.ShapeDtypeStruct((2, n_align, 128), jnp.float32)]
    if with_cnt:
        out_type.append(jax.ShapeDtypeStruct((2, n_align, 128), jnp.float32))
    scratch = [
        pltpu.VMEM_SHARED((n_align, 32), jnp.float32),
        pltpu.VMEM((nsteps, 128), jnp.int32),
        pltpu.VMEM((nsteps, 128), jnp.int32),
        pltpu.VMEM((8, 128, 32), jnp.float32),
        pltpu.VMEM((128, 32), jnp.float32),
        pltpu.SemaphoreType.DMA,
        pltpu.SemaphoreType.DMA,
    ]

    def body(*refs):
        if with_cnt:
            (h4, srcq, dst, out_s, out_c,
             acc, src_v, dst_v, rows_v, zbuf, sem, sem2) = refs
        else:
            (h4, srcq, dst, out_s,
             acc, src_v, dst_v, rows_v, zbuf, sem, sem2) = refs
            out_c = None
        cid = lax.axis_index("c")
        sid = lax.axis_index("s")
        wid = cid * 16 + sid
        r0 = sid * rows_pt
        z16 = jnp.zeros((16,), jnp.float32)

        def zb(i, c):
            zbuf[i, 0:16] = z16
            zbuf[i, 16:32] = z16
            return c
        lax.fori_loop(0, 128, zb, 0)
        pltpu.sync_copy(dst.at[wid], dst_v)
        npass = 5 if with_cnt else 4
        for p in range(npass):
            is_cnt = p == 4

            def zl(i, c):
                pltpu.async_copy(zbuf, acc.at[pl.ds(r0 + i * 128, 128)], sem)
                return c
            lax.fori_loop(0, nzero, zl, 0)

            def zl_drain(i, c):
                pltpu.make_async_copy(zbuf, acc.at[pl.ds(r0 + i * 128, 128)],
                                      sem).wait()
                return c
            lax.fori_loop(0, nzero, zl_drain, 0)
            if is_cnt:
                o16 = jnp.ones((16,), jnp.float32)

                def ob(i, c):
                    rows_v[0, i, 0:16] = o16
                    rows_v[0, i, 16:32] = o16
                    return c
                lax.fori_loop(0, 128, ob, 0)
            else:
                pltpu.sync_copy(srcq.at[p, wid], src_v)
            plsc.subcore_barrier()

            if is_cnt:
                def fire_c(j, c):
                    pltpu.async_copy(rows_v.at[0], acc.at[dst_v.at[j]], sem2,
                                     add=True)
                    return c
                lax.fori_loop(0, nsteps, fire_c, 0)

                def drain_c(j, c):
                    pltpu.make_async_copy(rows_v.at[0], acc.at[dst_v.at[j]],
                                          sem2).wait()
                    return c
                lax.fori_loop(0, nsteps, drain_c, 0)
            else:
                def chunk(base, nb):
                    for b in range(nb):
                        pltpu.async_copy(h4.at[src_v.at[base + b]],
                                         rows_v.at[b], sem)
                    for b in range(nb):
                        pltpu.make_async_copy(h4.at[src_v.at[base + b]],
                                              rows_v.at[b], sem).wait()
                        pltpu.async_copy(rows_v.at[b],
                                         acc.at[dst_v.at[base + b]], sem2,
                                         add=True)
                    for b in range(nb):
                        pltpu.make_async_copy(rows_v.at[b],
                                              acc.at[dst_v.at[base + b]],
                                              sem2).wait()

                nchunks, tail = divmod(nsteps, 8)

                def chunk_loop(c, x):
                    chunk(c * 8, 8)
                    return x
                lax.fori_loop(0, nchunks, chunk_loop, 0)
                if tail:
                    chunk(nchunks * 8, tail)
            plsc.subcore_barrier()
            if is_cnt:
                pltpu.sync_copy(acc.at[pl.ds(r0, rows_pt)],
                                out_c.at[cid, pl.ds(r0, rows_pt), pl.ds(0, 32)])
            else:
                pltpu.sync_copy(acc.at[pl.ds(r0, rows_pt)],
                                out_s.at[cid, pl.ds(r0, rows_pt),
                                         pl.ds(32 * p, 32)])

    return pl.kernel(
        body, out_type=out_type, mesh=mesh, scratch_types=scratch,
        compiler_params=pltpu.CompilerParams(use_tc_tiling_on_sc=False))


def _prep_edges(ei, n_dst, e_pad):
    e = ei.shape[1]
    nsteps = e_pad // 32 // 128
    srcq = (ei[0] * 4)[None, :] + jnp.arange(4, dtype=jnp.int32)[:, None]
    srcq = jnp.pad(srcq, ((0, 0), (0, e_pad - e)))
    dst = jnp.pad(ei[1], (0, e_pad - e), constant_values=n_dst)
    return (srcq.reshape(4, 32, nsteps, 128), dst.reshape(32, nsteps, 128))


def _segment_sums(h, prepped, with_cnt):
    sums, cnts = {}, {}
    for s, d, e in EDGE_TYPES:
        srcq, dst = prepped[(s, d)]
        h4 = h[s].reshape(4 * N_NODES[s], 32)
        fn = _make_sc_agg(_epad(e), _NALIGN[d], with_cnt)
        res = fn(h4, srcq, dst)
        if with_cnt:
            sums[(s, d)], cnts[(s, d)] = res
        else:
            sums[(s, d)] = res[0] if isinstance(res, (tuple, list)) else res
    return sums, cnts


def _hetero_layer(h, convp, prepped, cnts, want_stats, with_cnt=False):
    sums, new_cnts = _segment_sums(h, prepped, with_cnt)
    if with_cnt:
        cnts = new_cnts
    incoming = {}
    for s, d, _ in EDGE_TYPES:
        incoming.setdefault(d, []).append((s, d))
    out = {}
    for d, keys in incoming.items():
        p = [convp[s + '__' + dd] for s, dd in keys]
        y, st = _combine([sums[k] for k in keys], [cnts[k] for k in keys],
                         h[d], [q['wl'] for q in p], [q['bl'] for q in p],
                         [q['wr'] for q in p], want_stats)
        out[d] = (y, st)
    return out, cnts


def kernel(x_Patient, x_Admission, params, ei_Patient_Admission, ei_Admission_Patient, ei_Admission_Diagnosis, ei_Diagnosis_Admission, ei_Admission_Medication, ei_Medication_Admission, ei_Admission_Procedure, ei_Procedure_Admission, ei_Admission_LabTest, ei_LabTest_Admission):
    eis = {
        ('Patient', 'Admission'): ei_Patient_Admission,
        ('Admission', 'Patient'): ei_Admission_Patient,
        ('Admission', 'Diagnosis'): ei_Admission_Diagnosis,
        ('Diagnosis', 'Admission'): ei_Diagnosis_Admission,
        ('Admission', 'Medication'): ei_Admission_Medication,
        ('Medication', 'Admission'): ei_Medication_Admission,
        ('Admission', 'Procedure'): ei_Admission_Procedure,
        ('Procedure', 'Admission'): ei_Procedure_Admission,
        ('Admission', 'LabTest'): ei_Admission_LabTest,
        ('LabTest', 'Admission'): ei_LabTest_Admission,
    }
    enc = params['enc']
    h = {}
    yp, stp = _encode(x_Patient, enc['pat_w'], enc['pat_b'])
    h['Patient'] = _bn_apply(yp, stp, enc['pat_g'], enc['pat_be'], relu=False)
    ya, sta = _encode(x_Admission, enc['adm_w'], enc['adm_b'])
    h['Admission'] = _bn_apply(ya, sta, enc['adm_g'], enc['adm_be'], relu=False)
    for t in EMB_TYPES:
        h[t] = enc['emb_' + t]

    prepped = {(s, d): _prep_edges(eis[(s, d)], N_NODES[d], _epad(e))
               for s, d, e in EDGE_TYPES}

    cnts = None
    for L, bn in (('conv1', 'bn1'), ('conv2', 'bn2')):
        res, cnts = _hetero_layer(h, params[L], prepped, cnts,
                                  want_stats=True, with_cnt=(L == 'conv1'))
        h = {t: _bn_apply(res[t][0], res[t][1], params[bn][t]['g'],
                          params[bn][t]['b'], relu=True)
             for t in NODE_TYPES}

    res, _ = _hetero_layer(h, params['conv3'], prepped, cnts, want_stats=False)
    out = [_final(res[t][0], params['lin'][t]['w'], params['lin'][t]['b'])
           for t in NODE_TYPES]
    return jnp.concatenate(out, axis=0)
